# Initial kernel scaffold; baseline (speedup 1.0000x reference)
import jax
import jax.numpy as jnp
from jax.experimental import pallas as pl


def kernel(x, nb, pca_w, pca_b, ln_g, ln_b, wq, wk, mlp_w, mlp_b):
    raise NotImplementedError("write your pallas kernel here")


# speedup vs baseline: 74.2421x; 74.2421x over previous
import functools

import numpy as np
import jax
import jax.numpy as jnp
from jax import lax
from jax.experimental import pallas as pl
from jax.experimental.pallas import tpu as pltpu
from jax.experimental.pallas import tpu_sc as plsc

_N = 10000
_M = 32
_K = 8
_DD = 16
_D = 128
_ROUTIT = 6
_NW = 32
_CH = 128

_NPAD = 10240

_S_np = np.zeros((_D, _K), np.float32)
for _d in range(_D):
    _S_np[_d, _d // _DD] = 1.0
_T_np = np.zeros((_D, _DD), np.float32)
for _d in range(_D):
    _T_np[_d, _d % _DD] = 1.0


_BNA = 1000


def _front_body(x_ref, pw_ref, pb_ref, lgt_ref, lbt_ref, wqbd_ref, wkbd_ref,
                s_ref, st_ref, tt_ref, xn_ref, acc_ref):
    x = x_ref[...]
    s = s_ref[...]
    st = st_ref[...]
    h = jnp.dot(x, pw_ref[...], preferred_element_type=jnp.float32,
                 precision=lax.Precision.HIGHEST) + pb_ref[...]
    mu = jnp.dot(h, s, preferred_element_type=jnp.float32,
                 precision=lax.Precision.HIGHEST) * (1.0 / _DD)
    mue = jnp.dot(mu, st, preferred_element_type=jnp.float32,
                 precision=lax.Precision.HIGHEST)
    hc = h - mue
    var = jnp.dot(hc * hc, s, preferred_element_type=jnp.float32,
                 precision=lax.Precision.HIGHEST) * (1.0 / _DD)
    inve = jnp.dot(lax.rsqrt(var + 1e-6), st,
                   preferred_element_type=jnp.float32,
                 precision=lax.Precision.HIGHEST)
    qn = hc * inve * lgt_ref[...] + lbt_ref[...]
    qp = jnp.dot(qn, wqbd_ref[...], preferred_element_type=jnp.float32,
                 precision=lax.Precision.HIGHEST) * 0.25
    kp = jnp.dot(h, wkbd_ref[...], preferred_element_type=jnp.float32,
                 precision=lax.Precision.HIGHEST)
    alj = []
    for j in range(_K):
        kp_jt = jnp.dot(kp * st[j:j + 1, :], tt_ref[...],
                        preferred_element_type=jnp.float32,
                 precision=lax.Precision.HIGHEST)
        alj.append(jnp.dot(qp * kp_jt, s, preferred_element_type=jnp.float32,
                 precision=lax.Precision.HIGHEST))
    mx = alj[0]
    for j in range(1, _K):
        mx = jnp.maximum(mx, alj[j])
    se = jnp.zeros_like(mx)
    dg = jnp.zeros_like(mx)
    ks = lax.broadcasted_iota(jnp.int32, (_BNA, _K), 1)
    for j in range(_K):
        e = jnp.exp(alj[j] - mx)
        se = se + e
        dg = dg + e * (ks == j).astype(jnp.float32)
    off = jnp.sum(1.0 - dg / se).reshape(1, 1)

    @pl.when(pl.program_id(0) == 0)
    def _():
        acc_ref[...] = jnp.zeros((1, 1), jnp.float32)

    acc_ref[...] += off

    r = jnp.maximum(h, 0.0)
    n2 = jnp.dot(r * r, s, preferred_element_type=jnp.float32,
                 precision=lax.Precision.HIGHEST)
    inv = 1.0 / jnp.maximum(jnp.sqrt(n2), 1e-12)
    xn_ref[...] = r * jnp.dot(inv, st, preferred_element_type=jnp.float32,
                 precision=lax.Precision.HIGHEST)


def _run_front(x, pca_w, pca_b, lgt, lbt, wqbd, wkbd, s, st, p_tile):
    nblk = _N // _BNA
    return pl.pallas_call(
        _front_body,
        grid=(nblk,),
        in_specs=[
            pl.BlockSpec((_BNA, _D), lambda i: (i, 0)),
            pl.BlockSpec((_D, _D), lambda i: (0, 0)),
            pl.BlockSpec((1, _D), lambda i: (0, 0)),
            pl.BlockSpec((1, _D), lambda i: (0, 0)),
            pl.BlockSpec((1, _D), lambda i: (0, 0)),
            pl.BlockSpec((_D, _D), lambda i: (0, 0)),
            pl.BlockSpec((_D, _D), lambda i: (0, 0)),
            pl.BlockSpec((_D, _K), lambda i: (0, 0)),
            pl.BlockSpec((_K, _D), lambda i: (0, 0)),
            pl.BlockSpec((_D, _D), lambda i: (0, 0)),
        ],
        out_specs=[
            pl.BlockSpec((_BNA, _D), lambda i: (i, 0)),
            pl.BlockSpec((1, 1), lambda i: (0, 0)),
        ],
        out_shape=[
            jax.ShapeDtypeStruct((_N, _D), jnp.float32),
            jax.ShapeDtypeStruct((1, 1), jnp.float32),
        ],
    )(x, pca_w, pca_b, lgt, lbt, wqbd, wkbd, s, st, p_tile)


def _sc_gather(table, idx3, dt):
    nw, n_chunks, ch = idx3.shape
    per_w = n_chunks * ch
    mesh = plsc.VectorSubcoreMesh(core_axis_name="c", subcore_axis_name="s",
                                  num_cores=2, num_subcores=16)

    @functools.partial(
        pl.kernel, mesh=mesh,
        out_type=jax.ShapeDtypeStruct((nw * per_w, dt), jnp.float32),
        scratch_types=[
            pltpu.VMEM((n_chunks, ch), jnp.int32),
            pltpu.VMEM((ch, dt), jnp.float32),
            pltpu.VMEM((ch, dt), jnp.float32),
            pltpu.SemaphoreType.DMA,
            pltpu.SemaphoreType.DMA,
        ],
    )
    def gk(table_hbm, idx_hbm, out_hbm, idx_v, buf0, buf1, sem0, sem1):
        wid = lax.axis_index("s") * 2 + lax.axis_index("c")
        base = wid * per_w
        pltpu.sync_copy(idx_hbm.at[wid], idx_v)
        pltpu.async_copy(table_hbm.at[idx_v.at[0]], buf0, sem0)

        @pl.loop(0, n_chunks, step=2)
        def _(j):
            j1 = j + 1
            pltpu.async_copy(table_hbm.at[idx_v.at[j1]], buf1, sem1)
            pltpu.make_async_copy(table_hbm.at[idx_v.at[j]], buf0, sem0).wait()
            pltpu.sync_copy(buf0, out_hbm.at[pl.ds(base + j * ch, ch)])

            @pl.when(j + 2 < n_chunks)
            def _():
                pltpu.async_copy(table_hbm.at[idx_v.at[j + 2]], buf0, sem0)

            pltpu.make_async_copy(table_hbm.at[idx_v.at[j1]], buf1, sem1).wait()
            pltpu.sync_copy(buf1, out_hbm.at[pl.ds(base + j1 * ch, ch)])

    return gk(table, idx3)


_BNR = 512


def _routing_body(last, z_ref, xn_ref, s_ref, st_ref, t_ref, *outs):
    x = xn_ref[...]
    s = s_ref[...]
    st = st_ref[...]

    def capsn(u):
        n2 = jnp.dot(u * u, s, preferred_element_type=jnp.float32,
                 precision=lax.Precision.HIGHEST)
        inv = 1.0 / jnp.maximum(jnp.sqrt(n2), 1e-12)
        return u * jnp.dot(inv, st, preferred_element_type=jnp.float32,
                 precision=lax.Precision.HIGHEST)

    def logits(zm, u):
        return jnp.dot(zm * u, s, preferred_element_type=jnp.float32,
                 precision=lax.Precision.HIGHEST)

    def accum(m, carry):
        u, acc = carry
        zm = z_ref[m]
        l = logits(zm, u)
        e = jnp.exp(l - jnp.max(l, axis=1, keepdims=True))
        p = e / jnp.sum(e, axis=1, keepdims=True)
        pe = jnp.dot(p, st, preferred_element_type=jnp.float32,
                 precision=lax.Precision.HIGHEST)
        return u, acc + zm * pe

    def mean_step(m, acc):
        return acc + z_ref[m]

    u = lax.fori_loop(0, _M, mean_step, jnp.zeros((_BNR, _D), jnp.float32))
    u = capsn(u * (1.0 / _K) + x)

    def iter_step(_, u):
        _, acc = lax.fori_loop(0, _M, accum,
                               (u, jnp.zeros((_BNR, _D), jnp.float32)))
        return capsn(acc + x)

    u = lax.fori_loop(0, _ROUTIT - 2, iter_step, u)
    _, acc = lax.fori_loop(0, _M, accum,
                           (u, jnp.zeros((_BNR, _D), jnp.float32)))
    ufin = acc + x
    if not last:
        outs[0][...] = capsn(jnp.maximum(ufin, 0.0))
        return
    outs[0][...] = ufin
    ks = lax.broadcasted_iota(jnp.int32, (_BNR, _K), 1)
    hsum = jnp.zeros((_BNR, _D), jnp.float32)
    for a in range(5):
        za = z_ref[a]
        l = logits(za, u)
        mx = jnp.max(l, axis=1, keepdims=True)
        eq = l == mx
        first = jnp.min(jnp.where(eq, ks, _K), axis=1, keepdims=True)
        oh = (ks == first).astype(jnp.float32)
        outs[1][:, a, :] = oh
        hsum = hsum + za * jnp.dot(oh, st,
                                   preferred_element_type=jnp.float32,
                 precision=lax.Precision.HIGHEST)
    outs[2][...] = jnp.dot(hsum, t_ref[...],
                           preferred_element_type=jnp.float32,
                 precision=lax.Precision.HIGHEST)


def _run_routing(z, xn, s, st, t, last):
    nblk = _NPAD // _BNR
    if last:
        out_specs = [
            pl.BlockSpec((_BNR, _D), lambda i: (i, 0)),
            pl.BlockSpec((_BNR, 5, _K), lambda i: (i, 0, 0)),
            pl.BlockSpec((_BNR, _D), lambda i: (i, 0)),
        ]
        out_shape = [
            jax.ShapeDtypeStruct((_NPAD, _D), jnp.float32),
            jax.ShapeDtypeStruct((_NPAD, 5, _K), jnp.float32),
            jax.ShapeDtypeStruct((_NPAD, _D), jnp.float32),
        ]
    else:
        out_specs = [pl.BlockSpec((_BNR, _D), lambda i: (i, 0))]
        out_shape = [jax.ShapeDtypeStruct((_NPAD, _D), jnp.float32)]
    return pl.pallas_call(
        functools.partial(_routing_body, last),
        grid=(nblk,),
        in_specs=[
            pl.BlockSpec((_M, _BNR, _D), lambda i: (0, i, 0)),
            pl.BlockSpec((_BNR, _D), lambda i: (i, 0)),
            pl.BlockSpec((_D, _K), lambda i: (0, 0)),
            pl.BlockSpec((_K, _D), lambda i: (0, 0)),
            pl.BlockSpec((_D, _D), lambda i: (0, 0)),
        ],
        out_specs=out_specs,
        out_shape=out_shape,
    )(z, xn, s, st, t)


_BNF = 1000


def _final_body(u_ref, oh_ref, hg_ref, st_ref, tt_ref, mw_ref, mb_ref,
                lp_ref, mo_ref):
    mp = jnp.zeros((_BNF, _D), jnp.float32)
    for a in range(5):
        ohe = jnp.dot(oh_ref[:, a, :], st_ref[...],
                      preferred_element_type=jnp.float32,
                 precision=lax.Precision.HIGHEST)
        hgt = jnp.dot(hg_ref[:, a, :], tt_ref[...],
                      preferred_element_type=jnp.float32,
                 precision=lax.Precision.HIGHEST)
        mp = mp + ohe * hgt
    mh = u_ref[...] + mp * (1.0 / 25.0)
    mo = jnp.dot(jnp.maximum(mh, 0.0), mw_ref[...],
                 preferred_element_type=jnp.float32,
                 precision=lax.Precision.HIGHEST) + mb_ref[...]
    mx = jnp.max(mo, axis=1, keepdims=True)
    e = jnp.exp(mo - mx)
    lse = jnp.log(jnp.sum(e, axis=1, keepdims=True)) + mx
    lp_ref[...] = mo - lse
    mo_ref[...] = mo


def _run_final(u, oh, hg, st, tt, mlp_w, mlp_b):
    nblk = _N // _BNF
    return pl.pallas_call(
        _final_body,
        grid=(nblk,),
        in_specs=[
            pl.BlockSpec((_BNF, _D), lambda i: (i, 0)),
            pl.BlockSpec((_BNF, 5, _K), lambda i: (i, 0, 0)),
            pl.BlockSpec((_BNF, 5, _D), lambda i: (i, 0, 0)),
            pl.BlockSpec((_K, _D), lambda i: (0, 0)),
            pl.BlockSpec((_D, _D), lambda i: (0, 0)),
            pl.BlockSpec((_D, _DD), lambda i: (0, 0)),
            pl.BlockSpec((1, _DD), lambda i: (0, 0)),
        ],
        out_specs=[
            pl.BlockSpec((_BNF, _DD), lambda i: (i, 0)),
            pl.BlockSpec((_BNF, _DD), lambda i: (i, 0)),
        ],
        out_shape=[
            jax.ShapeDtypeStruct((_N, _DD), jnp.float32),
            jax.ShapeDtypeStruct((_N, _DD), jnp.float32),
        ],
    )(u, oh, hg, st, tt, mlp_w, mlp_b)


def kernel(x, nb, pca_w, pca_b, ln_g, ln_b, wq, wk, mlp_w, mlp_b):
    s = jnp.asarray(_S_np)
    st = jnp.asarray(_S_np.T.copy())
    t = jnp.asarray(np.pad(_T_np, ((0, 0), (0, _D - _DD))))
    tt = jnp.asarray(np.pad(_T_np.T, ((0, _D - _DD), (0, 0))))
    eye = jnp.eye(_K, dtype=jnp.float32)

    nb = nb.reshape(-1).astype(jnp.int32)
    nzc = (_NPAD * _M) // (_NW * _CH)
    nb_t = jnp.zeros((_M, _NPAD), jnp.int32).at[:, : _N].set(
        nb.reshape(_N, _M).T)
    idx_z = nb_t.reshape(_NW, nzc, _CH)
    nb5 = nb.reshape(_N, _M)[:, :5].reshape(-1)
    nhc = 14
    nb5_pad = jnp.zeros((_NW * nhc * _CH,), jnp.int32).at[: _N * 5].set(nb5)
    idx_h = nb5_pad.reshape(_NW, nhc, _CH)

    lgt = jnp.tile(ln_g, _K).reshape(1, _D)
    lbt = jnp.tile(ln_b, _K).reshape(1, _D)
    wqbd = jnp.kron(eye, wq)
    wkbd = jnp.kron(eye, wk)
    p_tile = jnp.asarray(_T_np @ _T_np.T)
    xn1, acc = _run_front(x, pca_w, pca_b.reshape(1, _D), lgt, lbt,
                          wqbd, wkbd, s, st, p_tile)
    attn_loss = acc[0, 0] / (56.0 * _N)

    xn1p = jnp.zeros((_NPAD, _D), jnp.float32).at[:_N].set(xn1)
    z1 = _sc_gather(xn1p, idx_z, _D).reshape(_M, _NPAD, _D)
    (xn2,) = _run_routing(z1, xn1p, s, st, t, last=False)

    z2 = _sc_gather(xn2, idx_z, _D).reshape(_M, _NPAD, _D)
    u2, oh, hh = _run_routing(z2, xn2, s, st, t, last=True)

    hg = _sc_gather(hh, idx_h, _D)[: _N * 5].reshape(_N, 5, _D)
    lp, mo = _run_final(u2[:_N], oh[:_N], hg, st, tt, mlp_w,
                        mlp_b.reshape(1, _DD))
    return lp, attn_loss, mo


# speedup vs baseline: 91.5281x; 1.2328x over previous
import functools

import numpy as np
import jax
import jax.numpy as jnp
from jax import lax
from jax.experimental import pallas as pl
from jax.experimental.pallas import tpu as pltpu
from jax.experimental.pallas import tpu_sc as plsc

_N = 10000
_M = 32
_K = 8
_DD = 16
_D = 128
_ROUTIT = 6
_NW = 32
_CH = 128

_NPAD = 10240

_S_np = np.zeros((_D, _K), np.float32)
for _d in range(_D):
    _S_np[_d, _d // _DD] = 1.0
_T_np = np.zeros((_D, _DD), np.float32)
for _d in range(_D):
    _T_np[_d, _d % _DD] = 1.0


_BNA = 1000


def _front_body(x_ref, pw_ref, pb_ref, lgt_ref, lbt_ref, wqbd_ref, wkbd_ref,
                s_ref, st_ref, tt_ref, xn_ref, acc_ref):
    x = x_ref[...]
    s = s_ref[...]
    st = st_ref[...]
    h = jnp.dot(x, pw_ref[...], preferred_element_type=jnp.float32,
                 precision=lax.Precision.HIGHEST) + pb_ref[...]
    mu = jnp.dot(h, s, preferred_element_type=jnp.float32,
                 precision=lax.Precision.HIGHEST) * (1.0 / _DD)
    mue = jnp.dot(mu, st, preferred_element_type=jnp.float32,
                 precision=lax.Precision.HIGHEST)
    hc = h - mue
    var = jnp.dot(hc * hc, s, preferred_element_type=jnp.float32,
                 precision=lax.Precision.HIGHEST) * (1.0 / _DD)
    inve = jnp.dot(lax.rsqrt(var + 1e-6), st,
                   preferred_element_type=jnp.float32,
                 precision=lax.Precision.HIGHEST)
    qn = hc * inve * lgt_ref[...] + lbt_ref[...]
    qp = jnp.dot(qn, wqbd_ref[...], preferred_element_type=jnp.float32,
                 precision=lax.Precision.HIGHEST) * 0.25
    kp = jnp.dot(h, wkbd_ref[...], preferred_element_type=jnp.float32,
                 precision=lax.Precision.HIGHEST)
    alj = []
    for j in range(_K):
        kp_jt = jnp.dot(kp * st[j:j + 1, :], tt_ref[...],
                        preferred_element_type=jnp.float32,
                 precision=lax.Precision.HIGHEST)
        alj.append(jnp.dot(qp * kp_jt, s, preferred_element_type=jnp.float32,
                 precision=lax.Precision.HIGHEST))
    mx = alj[0]
    for j in range(1, _K):
        mx = jnp.maximum(mx, alj[j])
    se = jnp.zeros_like(mx)
    dg = jnp.zeros_like(mx)
    ks = lax.broadcasted_iota(jnp.int32, (_BNA, _K), 1)
    for j in range(_K):
        e = jnp.exp(alj[j] - mx)
        se = se + e
        dg = dg + e * (ks == j).astype(jnp.float32)
    off = jnp.sum(1.0 - dg / se).reshape(1, 1)

    @pl.when(pl.program_id(0) == 0)
    def _():
        acc_ref[...] = jnp.zeros((1, 1), jnp.float32)

    acc_ref[...] += off

    r = jnp.maximum(h, 0.0)
    n2 = jnp.dot(r * r, s, preferred_element_type=jnp.float32,
                 precision=lax.Precision.HIGHEST)
    inv = 1.0 / jnp.maximum(jnp.sqrt(n2), 1e-12)
    xn_ref[...] = r * jnp.dot(inv, st, preferred_element_type=jnp.float32,
                 precision=lax.Precision.HIGHEST)


def _run_front(x, pca_w, pca_b, lgt, lbt, wqbd, wkbd, s, st, p_tile):
    nblk = _N // _BNA
    return pl.pallas_call(
        _front_body,
        grid=(nblk,),
        in_specs=[
            pl.BlockSpec((_BNA, _D), lambda i: (i, 0)),
            pl.BlockSpec((_D, _D), lambda i: (0, 0)),
            pl.BlockSpec((1, _D), lambda i: (0, 0)),
            pl.BlockSpec((1, _D), lambda i: (0, 0)),
            pl.BlockSpec((1, _D), lambda i: (0, 0)),
            pl.BlockSpec((_D, _D), lambda i: (0, 0)),
            pl.BlockSpec((_D, _D), lambda i: (0, 0)),
            pl.BlockSpec((_D, _K), lambda i: (0, 0)),
            pl.BlockSpec((_K, _D), lambda i: (0, 0)),
            pl.BlockSpec((_D, _D), lambda i: (0, 0)),
        ],
        out_specs=[
            pl.BlockSpec((_BNA, _D), lambda i: (i, 0)),
            pl.BlockSpec((1, 1), lambda i: (0, 0)),
        ],
        out_shape=[
            jax.ShapeDtypeStruct((_N, _D), jnp.float32),
            jax.ShapeDtypeStruct((1, 1), jnp.float32),
        ],
    )(x, pca_w, pca_b, lgt, lbt, wqbd, wkbd, s, st, p_tile)


def _sc_gather(table, idx3, dt):
    nw, n_chunks, ch = idx3.shape
    per_w = n_chunks * ch
    mesh = plsc.VectorSubcoreMesh(core_axis_name="c", subcore_axis_name="s",
                                  num_cores=2, num_subcores=16)

    @functools.partial(
        pl.kernel, mesh=mesh,
        out_type=jax.ShapeDtypeStruct((nw * per_w, dt), jnp.float32),
        scratch_types=[
            pltpu.VMEM((n_chunks, ch), jnp.int32),
            pltpu.VMEM((ch, dt), jnp.float32),
            pltpu.VMEM((ch, dt), jnp.float32),
            pltpu.SemaphoreType.DMA,
            pltpu.SemaphoreType.DMA,
        ],
    )
    def gk(table_hbm, idx_hbm, out_hbm, idx_v, buf0, buf1, sem0, sem1):
        wid = lax.axis_index("s") * 2 + lax.axis_index("c")
        base = wid * per_w
        pltpu.sync_copy(idx_hbm.at[wid], idx_v)
        pltpu.async_copy(table_hbm.at[idx_v.at[0]], buf0, sem0)

        @pl.loop(0, n_chunks, step=2)
        def _(j):
            j1 = j + 1
            pltpu.async_copy(table_hbm.at[idx_v.at[j1]], buf1, sem1)
            pltpu.make_async_copy(table_hbm.at[idx_v.at[j]], buf0, sem0).wait()
            pltpu.sync_copy(buf0, out_hbm.at[pl.ds(base + j * ch, ch)])

            @pl.when(j + 2 < n_chunks)
            def _():
                pltpu.async_copy(table_hbm.at[idx_v.at[j + 2]], buf0, sem0)

            pltpu.make_async_copy(table_hbm.at[idx_v.at[j1]], buf1, sem1).wait()
            pltpu.sync_copy(buf1, out_hbm.at[pl.ds(base + j1 * ch, ch)])

    return gk(table, idx3)


_BNR = 512
_MC = 4


def _routing_body(last, z_ref, xn_ref, s_ref, st_ref, t_ref, *outs):
    x = xn_ref[...]
    s = s_ref[...]
    st = st_ref[...]

    def capsn(u):
        n2 = jnp.dot(u * u, s, preferred_element_type=jnp.float32,
                 precision=lax.Precision.HIGHEST)
        inv = 1.0 / jnp.maximum(jnp.sqrt(n2), 1e-12)
        return u * jnp.dot(inv, st, preferred_element_type=jnp.float32,
                 precision=lax.Precision.HIGHEST)

    def logits(zm, u):
        return jnp.dot(zm * u, s, preferred_element_type=jnp.float32,
                 precision=lax.Precision.HIGHEST)

    def accum(c, carry):
        u, acc = carry
        zc = z_ref[pl.ds(c * _MC, _MC)]
        zu = (zc * u[None, :, :]).reshape(_MC * _BNR, _D)
        l = jnp.dot(zu, s, preferred_element_type=jnp.float32,
                    precision=lax.Precision.HIGHEST)
        e = jnp.exp(l - jnp.max(l, axis=1, keepdims=True))
        p = e / jnp.sum(e, axis=1, keepdims=True)
        pe = jnp.dot(p, st, preferred_element_type=jnp.float32,
                     precision=lax.Precision.HIGHEST)
        return u, acc + jnp.sum(zc * pe.reshape(_MC, _BNR, _D), axis=0)

    def mean_step(c, acc):
        return acc + jnp.sum(z_ref[pl.ds(c * _MC, _MC)], axis=0)

    nchunk = _M // _MC
    u = lax.fori_loop(0, nchunk, mean_step,
                      jnp.zeros((_BNR, _D), jnp.float32))
    u = capsn(u * (1.0 / _K) + x)

    def iter_step(_, u):
        _, acc = lax.fori_loop(0, nchunk, accum,
                               (u, jnp.zeros((_BNR, _D), jnp.float32)))
        return capsn(acc + x)

    u = lax.fori_loop(0, _ROUTIT - 2, iter_step, u)
    _, acc = lax.fori_loop(0, nchunk, accum,
                           (u, jnp.zeros((_BNR, _D), jnp.float32)))
    ufin = acc + x
    if not last:
        outs[0][...] = capsn(jnp.maximum(ufin, 0.0))
        return
    outs[0][...] = ufin
    ks = lax.broadcasted_iota(jnp.int32, (_BNR, _K), 1)
    hsum = jnp.zeros((_BNR, _D), jnp.float32)
    for a in range(5):
        za = z_ref[a]
        l = logits(za, u)
        mx = jnp.max(l, axis=1, keepdims=True)
        eq = l == mx
        first = jnp.min(jnp.where(eq, ks, _K), axis=1, keepdims=True)
        oh = (ks == first).astype(jnp.float32)
        outs[1][:, a, :] = oh
        hsum = hsum + za * jnp.dot(oh, st,
                                   preferred_element_type=jnp.float32,
                 precision=lax.Precision.HIGHEST)
    outs[2][...] = jnp.dot(hsum, t_ref[...],
                           preferred_element_type=jnp.float32,
                 precision=lax.Precision.HIGHEST)


def _run_routing(z, xn, s, st, t, last):
    nblk = _NPAD // _BNR
    if last:
        out_specs = [
            pl.BlockSpec((_BNR, _D), lambda i: (i, 0)),
            pl.BlockSpec((_BNR, 5, _K), lambda i: (i, 0, 0)),
            pl.BlockSpec((_BNR, _D), lambda i: (i, 0)),
        ]
        out_shape = [
            jax.ShapeDtypeStruct((_NPAD, _D), jnp.float32),
            jax.ShapeDtypeStruct((_NPAD, 5, _K), jnp.float32),
            jax.ShapeDtypeStruct((_NPAD, _D), jnp.float32),
        ]
    else:
        out_specs = [pl.BlockSpec((_BNR, _D), lambda i: (i, 0))]
        out_shape = [jax.ShapeDtypeStruct((_NPAD, _D), jnp.float32)]
    return pl.pallas_call(
        functools.partial(_routing_body, last),
        grid=(nblk,),
        in_specs=[
            pl.BlockSpec((_M, _BNR, _D), lambda i: (0, i, 0)),
            pl.BlockSpec((_BNR, _D), lambda i: (i, 0)),
            pl.BlockSpec((_D, _K), lambda i: (0, 0)),
            pl.BlockSpec((_K, _D), lambda i: (0, 0)),
            pl.BlockSpec((_D, _D), lambda i: (0, 0)),
        ],
        out_specs=out_specs,
        out_shape=out_shape,
    )(z, xn, s, st, t)


_BNF = 1000


def _final_body(u_ref, oh_ref, hg_ref, st_ref, tt_ref, mw_ref, mb_ref,
                lp_ref, mo_ref):
    mp = jnp.zeros((_BNF, _D), jnp.float32)
    for a in range(5):
        ohe = jnp.dot(oh_ref[:, a, :], st_ref[...],
                      preferred_element_type=jnp.float32,
                 precision=lax.Precision.HIGHEST)
        hgt = jnp.dot(hg_ref[:, a, :], tt_ref[...],
                      preferred_element_type=jnp.float32,
                 precision=lax.Precision.HIGHEST)
        mp = mp + ohe * hgt
    mh = u_ref[...] + mp * (1.0 / 25.0)
    mo = jnp.dot(jnp.maximum(mh, 0.0), mw_ref[...],
                 preferred_element_type=jnp.float32,
                 precision=lax.Precision.HIGHEST) + mb_ref[...]
    mx = jnp.max(mo, axis=1, keepdims=True)
    e = jnp.exp(mo - mx)
    lse = jnp.log(jnp.sum(e, axis=1, keepdims=True)) + mx
    lp_ref[...] = mo - lse
    mo_ref[...] = mo


def _run_final(u, oh, hg, st, tt, mlp_w, mlp_b):
    nblk = _N // _BNF
    return pl.pallas_call(
        _final_body,
        grid=(nblk,),
        in_specs=[
            pl.BlockSpec((_BNF, _D), lambda i: (i, 0)),
            pl.BlockSpec((_BNF, 5, _K), lambda i: (i, 0, 0)),
            pl.BlockSpec((_BNF, 5, _D), lambda i: (i, 0, 0)),
            pl.BlockSpec((_K, _D), lambda i: (0, 0)),
            pl.BlockSpec((_D, _D), lambda i: (0, 0)),
            pl.BlockSpec((_D, _DD), lambda i: (0, 0)),
            pl.BlockSpec((1, _DD), lambda i: (0, 0)),
        ],
        out_specs=[
            pl.BlockSpec((_BNF, _DD), lambda i: (i, 0)),
            pl.BlockSpec((_BNF, _DD), lambda i: (i, 0)),
        ],
        out_shape=[
            jax.ShapeDtypeStruct((_N, _DD), jnp.float32),
            jax.ShapeDtypeStruct((_N, _DD), jnp.float32),
        ],
    )(u, oh, hg, st, tt, mlp_w, mlp_b)


def kernel(x, nb, pca_w, pca_b, ln_g, ln_b, wq, wk, mlp_w, mlp_b):
    s = jnp.asarray(_S_np)
    st = jnp.asarray(_S_np.T.copy())
    t = jnp.asarray(np.pad(_T_np, ((0, 0), (0, _D - _DD))))
    tt = jnp.asarray(np.pad(_T_np.T, ((0, _D - _DD), (0, 0))))
    eye = jnp.eye(_K, dtype=jnp.float32)

    nb = nb.reshape(-1).astype(jnp.int32)
    nzc = (_NPAD * _M) // (_NW * _CH)
    nb_t = jnp.zeros((_M, _NPAD), jnp.int32).at[:, : _N].set(
        nb.reshape(_N, _M).T)
    idx_z = nb_t.reshape(_NW, nzc, _CH)
    nb5 = nb.reshape(_N, _M)[:, :5].reshape(-1)
    nhc = 14
    nb5_pad = jnp.zeros((_NW * nhc * _CH,), jnp.int32).at[: _N * 5].set(nb5)
    idx_h = nb5_pad.reshape(_NW, nhc, _CH)

    lgt = jnp.tile(ln_g, _K).reshape(1, _D)
    lbt = jnp.tile(ln_b, _K).reshape(1, _D)
    wqbd = jnp.kron(eye, wq)
    wkbd = jnp.kron(eye, wk)
    p_tile = jnp.asarray(_T_np @ _T_np.T)
    xn1, acc = _run_front(x, pca_w, pca_b.reshape(1, _D), lgt, lbt,
                          wqbd, wkbd, s, st, p_tile)
    attn_loss = acc[0, 0] / (56.0 * _N)

    xn1p = jnp.zeros((_NPAD, _D), jnp.float32).at[:_N].set(xn1)
    z1 = _sc_gather(xn1p, idx_z, _D).reshape(_M, _NPAD, _D)
    (xn2,) = _run_routing(z1, xn1p, s, st, t, last=False)

    z2 = _sc_gather(xn2, idx_z, _D).reshape(_M, _NPAD, _D)
    u2, oh, hh = _run_routing(z2, xn2, s, st, t, last=True)

    hg = _sc_gather(hh, idx_h, _D)[: _N * 5].reshape(_N, 5, _D)
    lp, mo = _run_final(u2[:_N], oh[:_N], hg, st, tt, mlp_w,
                        mlp_b.reshape(1, _DD))
    return lp, attn_loss, mo


# speedup vs baseline: 96.4488x; 1.0538x over previous
import functools

import numpy as np
import jax
import jax.numpy as jnp
from jax import lax
from jax.experimental import pallas as pl
from jax.experimental.pallas import tpu as pltpu
from jax.experimental.pallas import tpu_sc as plsc

_N = 10000
_M = 32
_K = 8
_DD = 16
_D = 128
_ROUTIT = 6
_NW = 32
_CH = 128

_NPAD = 10240

_S_np = np.zeros((_D, _K), np.float32)
for _d in range(_D):
    _S_np[_d, _d // _DD] = 1.0
_T_np = np.zeros((_D, _DD), np.float32)
for _d in range(_D):
    _T_np[_d, _d % _DD] = 1.0


_BNA = 1000


def _front_body(x_ref, pw_ref, pb_ref, lgt_ref, lbt_ref, wqbd_ref, wkbd_ref,
                s_ref, st_ref, tt_ref, xn_ref, acc_ref):
    x = x_ref[...]
    s = s_ref[...]
    st = st_ref[...]
    h = jnp.dot(x, pw_ref[...], preferred_element_type=jnp.float32,
                 precision=lax.Precision.HIGHEST) + pb_ref[...]
    mu = jnp.dot(h, s, preferred_element_type=jnp.float32,
                 precision=lax.Precision.HIGHEST) * (1.0 / _DD)
    mue = jnp.dot(mu, st, preferred_element_type=jnp.float32,
                 precision=lax.Precision.HIGHEST)
    hc = h - mue
    var = jnp.dot(hc * hc, s, preferred_element_type=jnp.float32,
                 precision=lax.Precision.HIGHEST) * (1.0 / _DD)
    inve = jnp.dot(lax.rsqrt(var + 1e-6), st,
                   preferred_element_type=jnp.float32,
                 precision=lax.Precision.HIGHEST)
    qn = hc * inve * lgt_ref[...] + lbt_ref[...]
    qp = jnp.dot(qn, wqbd_ref[...], preferred_element_type=jnp.float32) * 0.25
    kp = jnp.dot(h, wkbd_ref[...], preferred_element_type=jnp.float32)
    alj = []
    for j in range(_K):
        kp_jt = jnp.dot(kp * st[j:j + 1, :], tt_ref[...],
                        preferred_element_type=jnp.float32)
        alj.append(jnp.dot(qp * kp_jt, s,
                           preferred_element_type=jnp.float32))
    mx = alj[0]
    for j in range(1, _K):
        mx = jnp.maximum(mx, alj[j])
    se = jnp.zeros_like(mx)
    dg = jnp.zeros_like(mx)
    ks = lax.broadcasted_iota(jnp.int32, (_BNA, _K), 1)
    for j in range(_K):
        e = jnp.exp(alj[j] - mx)
        se = se + e
        dg = dg + e * (ks == j).astype(jnp.float32)
    off = jnp.sum(1.0 - dg / se).reshape(1, 1)

    @pl.when(pl.program_id(0) == 0)
    def _():
        acc_ref[...] = jnp.zeros((1, 1), jnp.float32)

    acc_ref[...] += off

    r = jnp.maximum(h, 0.0)
    n2 = jnp.dot(r * r, s, preferred_element_type=jnp.float32,
                 precision=lax.Precision.HIGHEST)
    inv = 1.0 / jnp.maximum(jnp.sqrt(n2), 1e-12)
    xn_ref[...] = r * jnp.dot(inv, st, preferred_element_type=jnp.float32,
                 precision=lax.Precision.HIGHEST)


def _run_front(x, pca_w, pca_b, lgt, lbt, wqbd, wkbd, s, st, p_tile):
    nblk = _N // _BNA
    return pl.pallas_call(
        _front_body,
        grid=(nblk,),
        in_specs=[
            pl.BlockSpec((_BNA, _D), lambda i: (i, 0)),
            pl.BlockSpec((_D, _D), lambda i: (0, 0)),
            pl.BlockSpec((1, _D), lambda i: (0, 0)),
            pl.BlockSpec((1, _D), lambda i: (0, 0)),
            pl.BlockSpec((1, _D), lambda i: (0, 0)),
            pl.BlockSpec((_D, _D), lambda i: (0, 0)),
            pl.BlockSpec((_D, _D), lambda i: (0, 0)),
            pl.BlockSpec((_D, _K), lambda i: (0, 0)),
            pl.BlockSpec((_K, _D), lambda i: (0, 0)),
            pl.BlockSpec((_D, _D), lambda i: (0, 0)),
        ],
        out_specs=[
            pl.BlockSpec((_BNA, _D), lambda i: (i, 0)),
            pl.BlockSpec((1, 1), lambda i: (0, 0)),
        ],
        out_shape=[
            jax.ShapeDtypeStruct((_N, _D), jnp.float32),
            jax.ShapeDtypeStruct((1, 1), jnp.float32),
        ],
    )(x, pca_w, pca_b, lgt, lbt, wqbd, wkbd, s, st, p_tile)


def _sc_gather(table, idx3, dt):
    nw, n_chunks, ch = idx3.shape
    per_w = n_chunks * ch
    mesh = plsc.VectorSubcoreMesh(core_axis_name="c", subcore_axis_name="s",
                                  num_cores=2, num_subcores=16)

    @functools.partial(
        pl.kernel, mesh=mesh,
        out_type=jax.ShapeDtypeStruct((nw * per_w, dt), jnp.float32),
        scratch_types=[
            pltpu.VMEM((n_chunks, ch), jnp.int32),
            pltpu.VMEM((ch, dt), jnp.float32),
            pltpu.VMEM((ch, dt), jnp.float32),
            pltpu.SemaphoreType.DMA,
            pltpu.SemaphoreType.DMA,
        ],
    )
    def gk(table_hbm, idx_hbm, out_hbm, idx_v, buf0, buf1, sem0, sem1):
        wid = lax.axis_index("s") * 2 + lax.axis_index("c")
        base = wid * per_w
        pltpu.sync_copy(idx_hbm.at[wid], idx_v)
        pltpu.async_copy(table_hbm.at[idx_v.at[0]], buf0, sem0)

        @pl.loop(0, n_chunks, step=2)
        def _(j):
            j1 = j + 1
            pltpu.async_copy(table_hbm.at[idx_v.at[j1]], buf1, sem1)
            pltpu.make_async_copy(table_hbm.at[idx_v.at[j]], buf0, sem0).wait()
            pltpu.sync_copy(buf0, out_hbm.at[pl.ds(base + j * ch, ch)])

            @pl.when(j + 2 < n_chunks)
            def _():
                pltpu.async_copy(table_hbm.at[idx_v.at[j + 2]], buf0, sem0)

            pltpu.make_async_copy(table_hbm.at[idx_v.at[j1]], buf1, sem1).wait()
            pltpu.sync_copy(buf1, out_hbm.at[pl.ds(base + j1 * ch, ch)])

    return gk(table, idx3)


_BNR = 512
_MC = 8


def _routing_body(last, z_ref, xn_ref, s_ref, st_ref, t_ref, *outs):
    x = xn_ref[...]
    s = s_ref[...]
    st = st_ref[...]

    def capsn(u):
        n2 = jnp.dot(u * u, s, preferred_element_type=jnp.float32,
                 precision=lax.Precision.HIGHEST)
        inv = 1.0 / jnp.maximum(jnp.sqrt(n2), 1e-12)
        return u * jnp.dot(inv, st, preferred_element_type=jnp.float32,
                 precision=lax.Precision.HIGHEST)

    def logits(zm, u):
        return jnp.dot(zm * u, s, preferred_element_type=jnp.float32,
                 precision=lax.Precision.HIGHEST)

    def accum(c, carry):
        u, acc = carry
        zc = z_ref[pl.ds(c * _MC, _MC)]
        zu = (zc * u[None, :, :]).reshape(_MC * _BNR, _D)
        l = jnp.dot(zu, s, preferred_element_type=jnp.float32,
                    precision=lax.Precision.HIGHEST)
        e = jnp.exp(l - jnp.max(l, axis=1, keepdims=True))
        p = e / jnp.sum(e, axis=1, keepdims=True)
        pe = jnp.dot(p, st, preferred_element_type=jnp.float32,
                     precision=lax.Precision.HIGHEST)
        return u, acc + jnp.sum(zc * pe.reshape(_MC, _BNR, _D), axis=0)

    def mean_step(c, acc):
        return acc + jnp.sum(z_ref[pl.ds(c * _MC, _MC)], axis=0)

    nchunk = _M // _MC
    u = lax.fori_loop(0, nchunk, mean_step,
                      jnp.zeros((_BNR, _D), jnp.float32))
    u = capsn(u * (1.0 / _K) + x)

    def iter_step(_, u):
        _, acc = lax.fori_loop(0, nchunk, accum,
                               (u, jnp.zeros((_BNR, _D), jnp.float32)))
        return capsn(acc + x)

    u = lax.fori_loop(0, _ROUTIT - 2, iter_step, u)
    _, acc = lax.fori_loop(0, nchunk, accum,
                           (u, jnp.zeros((_BNR, _D), jnp.float32)))
    ufin = acc + x
    if not last:
        outs[0][...] = capsn(jnp.maximum(ufin, 0.0))
        return
    outs[0][...] = ufin
    ks = lax.broadcasted_iota(jnp.int32, (_BNR, _K), 1)
    hsum = jnp.zeros((_BNR, _D), jnp.float32)
    for a in range(5):
        za = z_ref[a]
        l = logits(za, u)
        mx = jnp.max(l, axis=1, keepdims=True)
        eq = l == mx
        first = jnp.min(jnp.where(eq, ks, _K), axis=1, keepdims=True)
        oh = (ks == first).astype(jnp.float32)
        outs[1][:, a, :] = oh
        hsum = hsum + za * jnp.dot(oh, st,
                                   preferred_element_type=jnp.float32,
                 precision=lax.Precision.HIGHEST)
    outs[2][...] = jnp.dot(hsum, t_ref[...],
                           preferred_element_type=jnp.float32,
                 precision=lax.Precision.HIGHEST)


def _run_routing(z, xn, s, st, t, last):
    nblk = _NPAD // _BNR
    if last:
        out_specs = [
            pl.BlockSpec((_BNR, _D), lambda i: (i, 0)),
            pl.BlockSpec((_BNR, 5, _K), lambda i: (i, 0, 0)),
            pl.BlockSpec((_BNR, _D), lambda i: (i, 0)),
        ]
        out_shape = [
            jax.ShapeDtypeStruct((_NPAD, _D), jnp.float32),
            jax.ShapeDtypeStruct((_NPAD, 5, _K), jnp.float32),
            jax.ShapeDtypeStruct((_NPAD, _D), jnp.float32),
        ]
    else:
        out_specs = [pl.BlockSpec((_BNR, _D), lambda i: (i, 0))]
        out_shape = [jax.ShapeDtypeStruct((_NPAD, _D), jnp.float32)]
    return pl.pallas_call(
        functools.partial(_routing_body, last),
        grid=(nblk,),
        in_specs=[
            pl.BlockSpec((_M, _BNR, _D), lambda i: (0, i, 0)),
            pl.BlockSpec((_BNR, _D), lambda i: (i, 0)),
            pl.BlockSpec((_D, _K), lambda i: (0, 0)),
            pl.BlockSpec((_K, _D), lambda i: (0, 0)),
            pl.BlockSpec((_D, _D), lambda i: (0, 0)),
        ],
        out_specs=out_specs,
        out_shape=out_shape,
    )(z, xn, s, st, t)


_BNF = 1000


def _final_body(u_ref, oh_ref, hg_ref, st_ref, tt_ref, mw_ref, mb_ref,
                lp_ref, mo_ref):
    mp = jnp.zeros((_BNF, _D), jnp.float32)
    for a in range(5):
        ohe = jnp.dot(oh_ref[:, a, :], st_ref[...],
                      preferred_element_type=jnp.float32,
                 precision=lax.Precision.HIGHEST)
        hgt = jnp.dot(hg_ref[:, a, :], tt_ref[...],
                      preferred_element_type=jnp.float32,
                 precision=lax.Precision.HIGHEST)
        mp = mp + ohe * hgt
    mh = u_ref[...] + mp * (1.0 / 25.0)
    mo = jnp.dot(jnp.maximum(mh, 0.0), mw_ref[...],
                 preferred_element_type=jnp.float32,
                 precision=lax.Precision.HIGHEST) + mb_ref[...]
    mx = jnp.max(mo, axis=1, keepdims=True)
    e = jnp.exp(mo - mx)
    lse = jnp.log(jnp.sum(e, axis=1, keepdims=True)) + mx
    lp_ref[...] = mo - lse
    mo_ref[...] = mo


def _run_final(u, oh, hg, st, tt, mlp_w, mlp_b):
    nblk = _N // _BNF
    return pl.pallas_call(
        _final_body,
        grid=(nblk,),
        in_specs=[
            pl.BlockSpec((_BNF, _D), lambda i: (i, 0)),
            pl.BlockSpec((_BNF, 5, _K), lambda i: (i, 0, 0)),
            pl.BlockSpec((_BNF, 5, _D), lambda i: (i, 0, 0)),
            pl.BlockSpec((_K, _D), lambda i: (0, 0)),
            pl.BlockSpec((_D, _D), lambda i: (0, 0)),
            pl.BlockSpec((_D, _DD), lambda i: (0, 0)),
            pl.BlockSpec((1, _DD), lambda i: (0, 0)),
        ],
        out_specs=[
            pl.BlockSpec((_BNF, _DD), lambda i: (i, 0)),
            pl.BlockSpec((_BNF, _DD), lambda i: (i, 0)),
        ],
        out_shape=[
            jax.ShapeDtypeStruct((_N, _DD), jnp.float32),
            jax.ShapeDtypeStruct((_N, _DD), jnp.float32),
        ],
    )(u, oh, hg, st, tt, mlp_w, mlp_b)


def kernel(x, nb, pca_w, pca_b, ln_g, ln_b, wq, wk, mlp_w, mlp_b):
    s = jnp.asarray(_S_np)
    st = jnp.asarray(_S_np.T.copy())
    t = jnp.asarray(np.pad(_T_np, ((0, 0), (0, _D - _DD))))
    tt = jnp.asarray(np.pad(_T_np.T, ((0, _D - _DD), (0, 0))))
    eye = jnp.eye(_K, dtype=jnp.float32)

    nb = nb.reshape(-1).astype(jnp.int32)
    nzc = (_NPAD * _M) // (_NW * _CH)
    nb_t = jnp.zeros((_M, _NPAD), jnp.int32).at[:, : _N].set(
        nb.reshape(_N, _M).T)
    idx_z = nb_t.reshape(_NW, nzc, _CH)
    nb5 = nb.reshape(_N, _M)[:, :5].reshape(-1)
    nhc = 14
    nb5_pad = jnp.zeros((_NW * nhc * _CH,), jnp.int32).at[: _N * 5].set(nb5)
    idx_h = nb5_pad.reshape(_NW, nhc, _CH)

    lgt = jnp.tile(ln_g, _K).reshape(1, _D)
    lbt = jnp.tile(ln_b, _K).reshape(1, _D)
    wqbd = jnp.kron(eye, wq)
    wkbd = jnp.kron(eye, wk)
    p_tile = jnp.asarray(_T_np @ _T_np.T)
    xn1, acc = _run_front(x, pca_w, pca_b.reshape(1, _D), lgt, lbt,
                          wqbd, wkbd, s, st, p_tile)
    attn_loss = acc[0, 0] / (56.0 * _N)

    xn1p = jnp.zeros((_NPAD, _D), jnp.float32).at[:_N].set(xn1)
    z1 = _sc_gather(xn1p, idx_z, _D).reshape(_M, _NPAD, _D)
    (xn2,) = _run_routing(z1, xn1p, s, st, t, last=False)

    z2 = _sc_gather(xn2, idx_z, _D).reshape(_M, _NPAD, _D)
    u2, oh, hh = _run_routing(z2, xn2, s, st, t, last=True)

    hg = _sc_gather(hh, idx_h, _D)[: _N * 5].reshape(_N, 5, _D)
    lp, mo = _run_final(u2[:_N], oh[:_N], hg, st, tt, mlp_w,
                        mlp_b.reshape(1, _DD))
    return lp, attn_loss, mo


# speedup vs baseline: 96.4854x; 1.0004x over previous
import functools

import numpy as np
import jax
import jax.numpy as jnp
from jax import lax
from jax.experimental import pallas as pl
from jax.experimental.pallas import tpu as pltpu
from jax.experimental.pallas import tpu_sc as plsc

_N = 10000
_M = 32
_K = 8
_DD = 16
_D = 128
_ROUTIT = 6
_NW = 32
_CH = 128

_NPAD = 10240

_S_np = np.zeros((_D, _K), np.float32)
for _d in range(_D):
    _S_np[_d, _d // _DD] = 1.0
_T_np = np.zeros((_D, _DD), np.float32)
for _d in range(_D):
    _T_np[_d, _d % _DD] = 1.0


_BNA = 1000


def _front_body(x_ref, pw_ref, pb_ref, lgt_ref, lbt_ref, wqbd_ref, wkbd_ref,
                s_ref, st_ref, tt_ref, xn_ref, acc_ref):
    x = x_ref[...]
    s = s_ref[...]
    st = st_ref[...]
    h = jnp.dot(x, pw_ref[...], preferred_element_type=jnp.float32,
                 precision=lax.Precision.HIGHEST) + pb_ref[...]
    mu = jnp.dot(h, s, preferred_element_type=jnp.float32,
                 precision=lax.Precision.HIGHEST) * (1.0 / _DD)
    mue = jnp.dot(mu, st, preferred_element_type=jnp.float32,
                 precision=lax.Precision.HIGHEST)
    hc = h - mue
    var = jnp.dot(hc * hc, s, preferred_element_type=jnp.float32,
                 precision=lax.Precision.HIGHEST) * (1.0 / _DD)
    inve = jnp.dot(lax.rsqrt(var + 1e-6), st,
                   preferred_element_type=jnp.float32,
                 precision=lax.Precision.HIGHEST)
    qn = hc * inve * lgt_ref[...] + lbt_ref[...]
    qp = jnp.dot(qn, wqbd_ref[...], preferred_element_type=jnp.float32) * 0.25
    kp = jnp.dot(h, wkbd_ref[...], preferred_element_type=jnp.float32)
    alj = []
    for j in range(_K):
        kp_jt = jnp.dot(kp * st[j:j + 1, :], tt_ref[...],
                        preferred_element_type=jnp.float32)
        alj.append(jnp.dot(qp * kp_jt, s,
                           preferred_element_type=jnp.float32))
    mx = alj[0]
    for j in range(1, _K):
        mx = jnp.maximum(mx, alj[j])
    se = jnp.zeros_like(mx)
    dg = jnp.zeros_like(mx)
    ks = lax.broadcasted_iota(jnp.int32, (_BNA, _K), 1)
    for j in range(_K):
        e = jnp.exp(alj[j] - mx)
        se = se + e
        dg = dg + e * (ks == j).astype(jnp.float32)
    off = jnp.sum(1.0 - dg / se).reshape(1, 1)

    @pl.when(pl.program_id(0) == 0)
    def _():
        acc_ref[...] = jnp.zeros((1, 1), jnp.float32)

    acc_ref[...] += off

    r = jnp.maximum(h, 0.0)
    n2 = jnp.dot(r * r, s, preferred_element_type=jnp.float32,
                 precision=lax.Precision.HIGHEST)
    inv = 1.0 / jnp.maximum(jnp.sqrt(n2), 1e-12)
    xn_ref[...] = r * jnp.dot(inv, st, preferred_element_type=jnp.float32,
                 precision=lax.Precision.HIGHEST)


def _run_front(x, pca_w, pca_b, lgt, lbt, wqbd, wkbd, s, st, p_tile):
    nblk = _N // _BNA
    return pl.pallas_call(
        _front_body,
        grid=(nblk,),
        in_specs=[
            pl.BlockSpec((_BNA, _D), lambda i: (i, 0)),
            pl.BlockSpec((_D, _D), lambda i: (0, 0)),
            pl.BlockSpec((1, _D), lambda i: (0, 0)),
            pl.BlockSpec((1, _D), lambda i: (0, 0)),
            pl.BlockSpec((1, _D), lambda i: (0, 0)),
            pl.BlockSpec((_D, _D), lambda i: (0, 0)),
            pl.BlockSpec((_D, _D), lambda i: (0, 0)),
            pl.BlockSpec((_D, _K), lambda i: (0, 0)),
            pl.BlockSpec((_K, _D), lambda i: (0, 0)),
            pl.BlockSpec((_D, _D), lambda i: (0, 0)),
        ],
        out_specs=[
            pl.BlockSpec((_BNA, _D), lambda i: (i, 0)),
            pl.BlockSpec((1, 1), lambda i: (0, 0)),
        ],
        out_shape=[
            jax.ShapeDtypeStruct((_N, _D), jnp.float32),
            jax.ShapeDtypeStruct((1, 1), jnp.float32),
        ],
    )(x, pca_w, pca_b, lgt, lbt, wqbd, wkbd, s, st, p_tile)


def _sc_gather(table, idx3, dt):
    nw, n_chunks, ch = idx3.shape
    per_w = n_chunks * ch
    mesh = plsc.VectorSubcoreMesh(core_axis_name="c", subcore_axis_name="s",
                                  num_cores=2, num_subcores=16)

    @functools.partial(
        pl.kernel, mesh=mesh,
        out_type=jax.ShapeDtypeStruct((nw * per_w, dt), jnp.float32),
        scratch_types=[
            pltpu.VMEM((n_chunks, ch), jnp.int32),
            pltpu.VMEM((ch, dt), jnp.float32),
            pltpu.VMEM((ch, dt), jnp.float32),
            pltpu.SemaphoreType.DMA,
            pltpu.SemaphoreType.DMA,
        ],
    )
    def gk(table_hbm, idx_hbm, out_hbm, idx_v, buf0, buf1, sem0, sem1):
        wid = lax.axis_index("s") * 2 + lax.axis_index("c")
        base = wid * per_w
        pltpu.sync_copy(idx_hbm.at[wid], idx_v)
        pltpu.async_copy(table_hbm.at[idx_v.at[0]], buf0, sem0)

        @pl.loop(0, n_chunks, step=2)
        def _(j):
            j1 = j + 1
            pltpu.async_copy(table_hbm.at[idx_v.at[j1]], buf1, sem1)
            pltpu.make_async_copy(table_hbm.at[idx_v.at[j]], buf0, sem0).wait()
            pltpu.sync_copy(buf0, out_hbm.at[pl.ds(base + j * ch, ch)])

            @pl.when(j + 2 < n_chunks)
            def _():
                pltpu.async_copy(table_hbm.at[idx_v.at[j + 2]], buf0, sem0)

            pltpu.make_async_copy(table_hbm.at[idx_v.at[j1]], buf1, sem1).wait()
            pltpu.sync_copy(buf1, out_hbm.at[pl.ds(base + j1 * ch, ch)])

    return gk(table, idx3)


_BNR = 512
_MC = 8


def _routing_body(last, z_ref, xn_ref, s_ref, st_ref, t_ref, *outs):
    x = xn_ref[...]
    s = s_ref[...]
    st = st_ref[...]

    def capsn(u):
        n2 = jnp.dot(u * u, s, preferred_element_type=jnp.float32,
                 precision=lax.Precision.HIGHEST)
        inv = 1.0 / jnp.maximum(jnp.sqrt(n2), 1e-12)
        return u * jnp.dot(inv, st, preferred_element_type=jnp.float32,
                 precision=lax.Precision.HIGHEST)

    def logits(zm, u):
        return jnp.dot(zm * u, s, preferred_element_type=jnp.float32,
                 precision=lax.Precision.HIGHEST)

    def accum(c, carry):
        u, acc = carry
        zc = z_ref[pl.ds(c * _MC, _MC)]
        zu = (zc * u[None, :, :]).reshape(_MC * _BNR, _D)
        l = jnp.dot(zu, s, preferred_element_type=jnp.float32,
                    precision=lax.Precision.HIGHEST)
        e = jnp.exp(l - jnp.max(l, axis=1, keepdims=True))
        p = e / jnp.sum(e, axis=1, keepdims=True)
        pe = jnp.dot(p, st, preferred_element_type=jnp.float32,
                     precision=lax.Precision.HIGHEST)
        return u, acc + jnp.sum(zc * pe.reshape(_MC, _BNR, _D), axis=0)

    def mean_step(c, acc):
        return acc + jnp.sum(z_ref[pl.ds(c * _MC, _MC)], axis=0)

    nchunk = _M // _MC
    u = lax.fori_loop(0, nchunk, mean_step,
                      jnp.zeros((_BNR, _D), jnp.float32))
    u = capsn(u * (1.0 / _K) + x)

    def iter_step(_, u):
        _, acc = lax.fori_loop(0, nchunk, accum,
                               (u, jnp.zeros((_BNR, _D), jnp.float32)))
        return capsn(acc + x)

    u = lax.fori_loop(0, _ROUTIT - 2, iter_step, u)
    _, acc = lax.fori_loop(0, nchunk, accum,
                           (u, jnp.zeros((_BNR, _D), jnp.float32)))
    ufin = acc + x
    if not last:
        outs[0][...] = capsn(jnp.maximum(ufin, 0.0))
        return
    outs[0][...] = ufin
    ks = lax.broadcasted_iota(jnp.int32, (_BNR, _K), 1)
    hsum = jnp.zeros((_BNR, _D), jnp.float32)
    for a in range(5):
        za = z_ref[a]
        l = logits(za, u)
        mx = jnp.max(l, axis=1, keepdims=True)
        eq = l == mx
        first = jnp.min(jnp.where(eq, ks, _K), axis=1, keepdims=True)
        oh = (ks == first).astype(jnp.float32)
        outs[1][:, a, :] = oh
        hsum = hsum + za * jnp.dot(oh, st,
                                   preferred_element_type=jnp.float32,
                 precision=lax.Precision.HIGHEST)
    outs[2][...] = jnp.dot(hsum, t_ref[...],
                           preferred_element_type=jnp.float32,
                 precision=lax.Precision.HIGHEST)


def _run_routing(z, xn, s, st, t, last):
    nblk = _NPAD // _BNR
    if last:
        out_specs = [
            pl.BlockSpec((_BNR, _D), lambda i: (i, 0)),
            pl.BlockSpec((_BNR, 5, _K), lambda i: (i, 0, 0)),
            pl.BlockSpec((_BNR, _D), lambda i: (i, 0)),
        ]
        out_shape = [
            jax.ShapeDtypeStruct((_NPAD, _D), jnp.float32),
            jax.ShapeDtypeStruct((_NPAD, 5, _K), jnp.float32),
            jax.ShapeDtypeStruct((_NPAD, _D), jnp.float32),
        ]
    else:
        out_specs = [pl.BlockSpec((_BNR, _D), lambda i: (i, 0))]
        out_shape = [jax.ShapeDtypeStruct((_NPAD, _D), jnp.float32)]
    return pl.pallas_call(
        functools.partial(_routing_body, last),
        grid=(nblk,),
        in_specs=[
            pl.BlockSpec((_M, _BNR, _D), lambda i: (0, i, 0)),
            pl.BlockSpec((_BNR, _D), lambda i: (i, 0)),
            pl.BlockSpec((_D, _K), lambda i: (0, 0)),
            pl.BlockSpec((_K, _D), lambda i: (0, 0)),
            pl.BlockSpec((_D, _D), lambda i: (0, 0)),
        ],
        out_specs=out_specs,
        out_shape=out_shape,
    )(z, xn, s, st, t)


_BNF = 1000


def _final_body(u_ref, oh_ref, hg_ref, st_ref, tt_ref, mw_ref, mb_ref,
                lp_ref, mo_ref):
    mp = jnp.zeros((_BNF, _D), jnp.float32)
    for a in range(5):
        ohe = jnp.dot(oh_ref[:, a, :], st_ref[...],
                      preferred_element_type=jnp.float32,
                 precision=lax.Precision.HIGHEST)
        hgt = jnp.dot(hg_ref[:, a, :], tt_ref[...],
                      preferred_element_type=jnp.float32,
                 precision=lax.Precision.HIGHEST)
        mp = mp + ohe * hgt
    mh = u_ref[...] + mp * (1.0 / 25.0)
    mo = jnp.dot(jnp.maximum(mh, 0.0), mw_ref[...],
                 preferred_element_type=jnp.float32,
                 precision=lax.Precision.HIGHEST) + mb_ref[...]
    mx = jnp.max(mo, axis=1, keepdims=True)
    e = jnp.exp(mo - mx)
    lse = jnp.log(jnp.sum(e, axis=1, keepdims=True)) + mx
    lp_ref[...] = mo - lse
    mo_ref[...] = mo


def _run_final(u, oh, hg, st, tt, mlp_w, mlp_b):
    nblk = _N // _BNF
    return pl.pallas_call(
        _final_body,
        grid=(nblk,),
        in_specs=[
            pl.BlockSpec((_BNF, _D), lambda i: (i, 0)),
            pl.BlockSpec((_BNF, 5, _K), lambda i: (i, 0, 0)),
            pl.BlockSpec((_BNF, 5, _D), lambda i: (i, 0, 0)),
            pl.BlockSpec((_K, _D), lambda i: (0, 0)),
            pl.BlockSpec((_D, _D), lambda i: (0, 0)),
            pl.BlockSpec((_D, _DD), lambda i: (0, 0)),
            pl.BlockSpec((1, _DD), lambda i: (0, 0)),
        ],
        out_specs=[
            pl.BlockSpec((_BNF, _DD), lambda i: (i, 0)),
            pl.BlockSpec((_BNF, _DD), lambda i: (i, 0)),
        ],
        out_shape=[
            jax.ShapeDtypeStruct((_N, _DD), jnp.float32),
            jax.ShapeDtypeStruct((_N, _DD), jnp.float32),
        ],
    )(u, oh, hg, st, tt, mlp_w, mlp_b)


def kernel(x, nb, pca_w, pca_b, ln_g, ln_b, wq, wk, mlp_w, mlp_b):
    s = jnp.asarray(_S_np)
    st = jnp.asarray(_S_np.T.copy())
    t = jnp.asarray(np.pad(_T_np, ((0, 0), (0, _D - _DD))))
    tt = jnp.asarray(np.pad(_T_np.T, ((0, _D - _DD), (0, 0))))
    eye = jnp.eye(_K, dtype=jnp.float32)

    nb = nb.reshape(-1).astype(jnp.int32)
    nzc = (_NPAD * _M) // (_NW * _CH)
    nb_t = jnp.zeros((_M, _NPAD), jnp.int32).at[:, : _N].set(
        nb.reshape(_N, _M).T)
    idx_z = nb_t.reshape(_NW, nzc, _CH)
    nb5 = nb.reshape(_N, _M)[:, :5].reshape(-1)
    nhc = -(-(_N * 5) // (_NW * _CH))
    nhc += nhc % 2
    nb5_pad = jnp.zeros((_NW * nhc * _CH,), jnp.int32).at[: _N * 5].set(nb5)
    idx_h = nb5_pad.reshape(_NW, nhc, _CH)

    lgt = jnp.tile(ln_g, _K).reshape(1, _D)
    lbt = jnp.tile(ln_b, _K).reshape(1, _D)
    wqbd = jnp.kron(eye, wq)
    wkbd = jnp.kron(eye, wk)
    p_tile = jnp.asarray(_T_np @ _T_np.T)
    xn1, acc = _run_front(x, pca_w, pca_b.reshape(1, _D), lgt, lbt,
                          wqbd, wkbd, s, st, p_tile)
    attn_loss = acc[0, 0] / (56.0 * _N)

    xn1p = jnp.zeros((_NPAD, _D), jnp.float32).at[:_N].set(xn1)
    z1 = _sc_gather(xn1p, idx_z, _D).reshape(_M, _NPAD, _D)
    (xn2,) = _run_routing(z1, xn1p, s, st, t, last=False)

    z2 = _sc_gather(xn2, idx_z, _D).reshape(_M, _NPAD, _D)
    u2, oh, hh = _run_routing(z2, xn2, s, st, t, last=True)

    hg = _sc_gather(hh, idx_h, _D)[: _N * 5].reshape(_N, 5, _D)
    lp, mo = _run_final(u2[:_N], oh[:_N], hg, st, tt, mlp_w,
                        mlp_b.reshape(1, _DD))
    return lp, attn_loss, mo
